# baseline (speedup 1.0000x reference)
import functools

import jax
import jax.numpy as jnp
from jax import lax
from jax.experimental import pallas as pl
from jax.experimental.pallas import tpu as pltpu
from jax.experimental.pallas import tpu_sc as plsc

_N, _C, _H, _W = 64, 384, 24, 24
_RPB = 2


def _tc_copy_min_body(x_ref, y_ref, mb_ref):
    i = pl.program_id(0)

    @pl.when(i == 0)
    def _init():
        mb_ref[...] = jnp.full((1, 128), jnp.inf, jnp.float32)

    v = x_ref[...]
    y_ref[...] = v
    mb_ref[...] = jnp.minimum(mb_ref[...], jnp.min(v))


_tc_pass = pl.pallas_call(
    _tc_copy_min_body,
    grid=(_N // _RPB,),
    in_specs=[pl.BlockSpec((_RPB, _C, _H, _W), lambda i: (i, 0, 0, 0))],
    out_specs=[
        pl.BlockSpec((_RPB, _C, _H, _W), lambda i: (i, 0, 0, 0)),
        pl.BlockSpec((1, 128), lambda i: (0, 0)),
    ],
    out_shape=[
        jax.ShapeDtypeStruct((_N, _C, _H, _W), jnp.float32),
        jax.ShapeDtypeStruct((1, 128), jnp.float32),
    ],
)

_sc_scratch = [
    pltpu.VMEM((_N, _H, _W), jnp.float32),
    pltpu.VMEM((_N,), jnp.int32),
    pltpu.VMEM((1, 128), jnp.float32),
    pltpu.SemaphoreType.DMA,
]


def _sc_scatter_body(y_hbm, idx_hbm, mb_hbm, vals_v, idx_v, mb_v, sem):
    cid = lax.axis_index("c")
    sid = lax.axis_index("s")

    @pl.when(jnp.logical_and(cid == 0, sid == 0))
    def _():
        pltpu.sync_copy(idx_hbm, idx_v)
        pltpu.sync_copy(mb_hbm, mb_v)

        acc = mb_v[0, pl.ds(0, 16)]
        for k in range(1, 8):
            acc = jnp.minimum(acc, mb_v[0, pl.ds(16 * k, 16)])
        m0 = jnp.min(acc)

        def body(i, m):
            val = jnp.where(m == 0.0, jnp.float32(0.0), m - jnp.float32(1e7))
            vv = jnp.full((16,), val, jnp.float32)
            for r in range(_H):
                vals_v[i, r, pl.ds(0, 16)] = vv
                vals_v[i, r, pl.ds(8, 16)] = vv
            return val

        lax.fori_loop(0, _N, body, m0)

        chunks = [idx_v[pl.ds(16 * a, 16)] for a in range(_N // 16)]
        copies = []
        for i in range(_N):
            ch = chunks[i // 16][i % 16]
            copies.append(pltpu.async_copy(vals_v.at[i], y_hbm.at[i, ch], sem))
        for cp in copies:
            cp.wait()


@functools.cache
def _get_sc_scatter():
    mesh = plsc.VectorSubcoreMesh(core_axis_name="c", subcore_axis_name="s")
    return pl.kernel(
        _sc_scatter_body,
        out_type=(),
        mesh=mesh,
        scratch_types=_sc_scratch,
        compiler_params=pltpu.CompilerParams(
            needs_layout_passes=False, use_tc_tiling_on_sc=False
        ),
    )


def kernel(x, indices):
    y, mb = _tc_pass(x)
    y_ref = jax.new_ref(y)
    _get_sc_scatter()(y_ref, indices, mb)
    return jax.freeze(y_ref)


# speedup vs baseline: 16.0648x; 16.0648x over previous
import functools

import jax
import jax.numpy as jnp
from jax import lax
from jax.experimental import pallas as pl
from jax.experimental.pallas import tpu as pltpu
from jax.experimental.pallas import tpu_sc as plsc

_N, _C, _H, _W = 64, 384, 24, 24
_HW = _H * _W
_RPB = 4
_LANES = 128
_NSUB = 32


def _tc_copy_min_body(x_ref, y_ref, mb_ref):
    i = pl.program_id(0)

    @pl.when(i == 0)
    def _init():
        mb_ref[...] = jnp.full((1, 128), jnp.inf, jnp.float32)

    v = x_ref[...]
    y_ref[...] = v
    mb_ref[...] = jnp.minimum(mb_ref[...], jnp.min(v))


_tc_pass = pl.pallas_call(
    _tc_copy_min_body,
    grid=(_N // _RPB,),
    in_specs=[pl.BlockSpec((_RPB, _H, _W, _C), lambda i: (i, 0, 0, 0))],
    out_specs=[
        pl.BlockSpec((_RPB, _H, _W, _C), lambda i: (i, 0, 0, 0)),
        pl.BlockSpec((1, 128), lambda i: (0, 0)),
    ],
    out_shape=[
        jax.ShapeDtypeStruct((_N, _H, _W, _C), jnp.float32),
        jax.ShapeDtypeStruct((1, 128), jnp.float32),
    ],
)

_sc_scratch = [
    pltpu.VMEM((_H, _W, _LANES), jnp.float32),
    pltpu.VMEM((_N,), jnp.int32),
    pltpu.VMEM((1, 128), jnp.float32),
    pltpu.SemaphoreType.DMA,
]


def _sc_scatter_body(y_hbm, idx_hbm, mb_hbm, tile_v, idx_v, mb_v, sem):
    cid = lax.axis_index("c")
    sid = lax.axis_index("s")
    wid = sid * 2 + cid

    pltpu.sync_copy(idx_hbm, idx_v)
    pltpu.sync_copy(mb_hbm, mb_v)

    acc = mb_v[0, pl.ds(0, 16)]
    for k in range(1, 8):
        acc = jnp.minimum(acc, mb_v[0, pl.ds(16 * k, 16)])
    m0 = jnp.min(acc)

    i0 = wid * 2
    i1 = i0 + 1

    def step(k, carry):
        m, v0, v1 = carry
        val = jnp.where(m == 0.0, jnp.float32(0.0), m - jnp.float32(1e7))
        v0 = jnp.where(k == i0, val, v0)
        v1 = jnp.where(k == i1, val, v1)
        return (val, v0, v1)

    _, v0, v1 = lax.fori_loop(0, _N, step, (m0, jnp.float32(0), jnp.float32(0)))

    chunks = [idx_v[pl.ds(16 * a, 16)] for a in range(_N // 16)]

    def channel_of(i):
        ch = jnp.int32(0)
        for k in range(_N):
            ch = jnp.where(i == k, chunks[k // 16][k % 16], ch)
        return ch

    base = lax.iota(jnp.int32, 16)
    rws = []
    for k in range(_HW // 16):
        rw = base + jnp.int32(16 * k)
        rws.append((rw // _W, rw % _W))

    def ablate(i, val):
        ch = channel_of(i)
        off = pl.multiple_of((ch // _LANES) * _LANES, _LANES)
        lane_vec = jnp.full((16,), ch % _LANES, jnp.int32)
        vv = jnp.full((16,), val, jnp.float32)
        dst = y_hbm.at[i, :, :, pl.ds(off, _LANES)]
        pltpu.sync_copy(dst, tile_v)
        for r_vec, w_vec in rws:
            plsc.store_scatter(tile_v, [r_vec, w_vec, lane_vec], vv)
        pltpu.sync_copy(tile_v, dst)

    ablate(i0, v0)
    ablate(i1, v1)


@functools.cache
def _get_sc_scatter():
    mesh = plsc.VectorSubcoreMesh(core_axis_name="c", subcore_axis_name="s")
    return pl.kernel(
        _sc_scatter_body,
        out_type=(),
        mesh=mesh,
        scratch_types=_sc_scratch,
        compiler_params=pltpu.CompilerParams(needs_layout_passes=False),
    )


def kernel(x, indices):
    xt = jnp.transpose(x, (0, 2, 3, 1))
    y, mb = _tc_pass(xt)
    y_ref = jax.new_ref(y)
    _get_sc_scatter()(y_ref, indices, mb)
    return jnp.transpose(jax.freeze(y_ref), (0, 3, 1, 2))


# speedup vs baseline: 16.6184x; 1.0345x over previous
import functools

import jax
import jax.numpy as jnp
from jax import lax
from jax.experimental import pallas as pl
from jax.experimental.pallas import tpu as pltpu
from jax.experimental.pallas import tpu_sc as plsc

_N, _C, _H, _W = 64, 384, 24, 24
_HW = _H * _W
_RPB = 8
_LANES = 128
_NSUB = 32


def _tc_copy_min_body(x_ref, y_ref, mb_ref):
    i = pl.program_id(0)

    @pl.when(i == 0)
    def _init():
        mb_ref[...] = jnp.full((1, 128), jnp.inf, jnp.float32)

    v = x_ref[...]
    y_ref[...] = v
    mb_ref[...] = jnp.minimum(mb_ref[...], jnp.min(v))


_tc_pass = pl.pallas_call(
    _tc_copy_min_body,
    grid=(_N // _RPB,),
    in_specs=[pl.BlockSpec((_RPB, _H, _W, _C), lambda i: (i, 0, 0, 0))],
    out_specs=[
        pl.BlockSpec((_RPB, _H, _W, _C), lambda i: (i, 0, 0, 0)),
        pl.BlockSpec((1, 128), lambda i: (0, 0)),
    ],
    out_shape=[
        jax.ShapeDtypeStruct((_N, _H, _W, _C), jnp.float32),
        jax.ShapeDtypeStruct((1, 128), jnp.float32),
    ],
)

_sc_scratch = [
    pltpu.VMEM((_H, _W, _LANES), jnp.float32),
    pltpu.VMEM((_N,), jnp.int32),
    pltpu.VMEM((1, 128), jnp.float32),
    pltpu.SemaphoreType.DMA,
]


def _sc_scatter_body(y_hbm, idx_hbm, mb_hbm, tile_v, idx_v, mb_v, sem):
    cid = lax.axis_index("c")
    sid = lax.axis_index("s")
    wid = sid * 2 + cid

    pltpu.sync_copy(idx_hbm, idx_v)
    pltpu.sync_copy(mb_hbm, mb_v)

    acc = mb_v[0, pl.ds(0, 16)]
    for k in range(1, 8):
        acc = jnp.minimum(acc, mb_v[0, pl.ds(16 * k, 16)])
    m0 = jnp.min(acc)

    i0 = wid * 2
    i1 = i0 + 1

    def step(k, carry):
        m, v0, v1 = carry
        val = jnp.where(m == 0.0, jnp.float32(0.0), m - jnp.float32(1e7))
        v0 = jnp.where(k == i0, val, v0)
        v1 = jnp.where(k == i1, val, v1)
        return (val, v0, v1)

    _, v0, v1 = lax.fori_loop(0, _N, step, (m0, jnp.float32(0), jnp.float32(0)))

    chunks = [idx_v[pl.ds(16 * a, 16)] for a in range(_N // 16)]

    def channel_of(i):
        ch = jnp.int32(0)
        for k in range(_N):
            ch = jnp.where(i == k, chunks[k // 16][k % 16], ch)
        return ch

    base = lax.iota(jnp.int32, 16)
    rws = []
    for k in range(_HW // 16):
        rw = base + jnp.int32(16 * k)
        rws.append((rw // _W, rw % _W))

    def ablate(i, val):
        ch = channel_of(i)
        off = pl.multiple_of((ch // _LANES) * _LANES, _LANES)
        lane_vec = jnp.full((16,), ch % _LANES, jnp.int32)
        vv = jnp.full((16,), val, jnp.float32)
        dst = y_hbm.at[i, :, :, pl.ds(off, _LANES)]
        pltpu.sync_copy(dst, tile_v)
        for r_vec, w_vec in rws:
            plsc.store_scatter(tile_v, [r_vec, w_vec, lane_vec], vv)
        pltpu.sync_copy(tile_v, dst)

    ablate(i0, v0)
    ablate(i1, v1)


@functools.cache
def _get_sc_scatter():
    mesh = plsc.VectorSubcoreMesh(core_axis_name="c", subcore_axis_name="s")
    return pl.kernel(
        _sc_scatter_body,
        out_type=(),
        mesh=mesh,
        scratch_types=_sc_scratch,
        compiler_params=pltpu.CompilerParams(needs_layout_passes=False),
    )


def kernel(x, indices):
    xt = jnp.transpose(x, (0, 2, 3, 1))
    y, mb = _tc_pass(xt)
    y_ref = jax.new_ref(y)
    _get_sc_scatter()(y_ref, indices, mb)
    return jnp.transpose(jax.freeze(y_ref), (0, 3, 1, 2))


# speedup vs baseline: 16.8038x; 1.0112x over previous
import functools

import jax
import jax.numpy as jnp
from jax import lax
from jax.experimental import pallas as pl
from jax.experimental.pallas import tpu as pltpu
from jax.experimental.pallas import tpu_sc as plsc

_N, _C, _H, _W = 64, 384, 24, 24
_HW = _H * _W
_RPB = 16
_LANES = 128
_NSUB = 32


def _tc_copy_min_body(x_ref, y_ref, mb_ref):
    i = pl.program_id(0)

    @pl.when(i == 0)
    def _init():
        mb_ref[...] = jnp.full((1, 128), jnp.inf, jnp.float32)

    v = x_ref[...]
    y_ref[...] = v
    mb_ref[...] = jnp.minimum(mb_ref[...], jnp.min(v))


_tc_pass = pl.pallas_call(
    _tc_copy_min_body,
    grid=(_N // _RPB,),
    in_specs=[pl.BlockSpec((_RPB, _H, _W, _C), lambda i: (i, 0, 0, 0))],
    out_specs=[
        pl.BlockSpec((_RPB, _H, _W, _C), lambda i: (i, 0, 0, 0)),
        pl.BlockSpec((1, 128), lambda i: (0, 0)),
    ],
    out_shape=[
        jax.ShapeDtypeStruct((_N, _H, _W, _C), jnp.float32),
        jax.ShapeDtypeStruct((1, 128), jnp.float32),
    ],
)

_sc_scratch = [
    pltpu.VMEM((_H, _W, _LANES), jnp.float32),
    pltpu.VMEM((_N,), jnp.int32),
    pltpu.VMEM((1, 128), jnp.float32),
    pltpu.SemaphoreType.DMA,
]


def _sc_scatter_body(y_hbm, idx_hbm, mb_hbm, tile_v, idx_v, mb_v, sem):
    cid = lax.axis_index("c")
    sid = lax.axis_index("s")
    wid = sid * 2 + cid

    pltpu.sync_copy(idx_hbm, idx_v)
    pltpu.sync_copy(mb_hbm, mb_v)

    acc = mb_v[0, pl.ds(0, 16)]
    for k in range(1, 8):
        acc = jnp.minimum(acc, mb_v[0, pl.ds(16 * k, 16)])
    m0 = jnp.min(acc)

    i0 = wid * 2
    i1 = i0 + 1

    def step(k, carry):
        m, v0, v1 = carry
        val = jnp.where(m == 0.0, jnp.float32(0.0), m - jnp.float32(1e7))
        v0 = jnp.where(k == i0, val, v0)
        v1 = jnp.where(k == i1, val, v1)
        return (val, v0, v1)

    _, v0, v1 = lax.fori_loop(0, _N, step, (m0, jnp.float32(0), jnp.float32(0)))

    chunks = [idx_v[pl.ds(16 * a, 16)] for a in range(_N // 16)]

    def channel_of(i):
        ch = jnp.int32(0)
        for k in range(_N):
            ch = jnp.where(i == k, chunks[k // 16][k % 16], ch)
        return ch

    base = lax.iota(jnp.int32, 16)
    rws = []
    for k in range(_HW // 16):
        rw = base + jnp.int32(16 * k)
        rws.append((rw // _W, rw % _W))

    def ablate(i, val):
        ch = channel_of(i)
        off = pl.multiple_of((ch // _LANES) * _LANES, _LANES)
        lane_vec = jnp.full((16,), ch % _LANES, jnp.int32)
        vv = jnp.full((16,), val, jnp.float32)
        dst = y_hbm.at[i, :, :, pl.ds(off, _LANES)]
        pltpu.sync_copy(dst, tile_v)
        for r_vec, w_vec in rws:
            plsc.store_scatter(tile_v, [r_vec, w_vec, lane_vec], vv)
        pltpu.sync_copy(tile_v, dst)

    ablate(i0, v0)
    ablate(i1, v1)


@functools.cache
def _get_sc_scatter():
    mesh = plsc.VectorSubcoreMesh(core_axis_name="c", subcore_axis_name="s")
    return pl.kernel(
        _sc_scatter_body,
        out_type=(),
        mesh=mesh,
        scratch_types=_sc_scratch,
        compiler_params=pltpu.CompilerParams(needs_layout_passes=False),
    )


def kernel(x, indices):
    xt = jnp.transpose(x, (0, 2, 3, 1))
    y, mb = _tc_pass(xt)
    y_ref = jax.new_ref(y)
    _get_sc_scatter()(y_ref, indices, mb)
    return jnp.transpose(jax.freeze(y_ref), (0, 3, 1, 2))
